# baseline (speedup 1.0000x reference)
import functools

import jax
import jax.numpy as jnp
from jax import lax
from jax.experimental import pallas as pl
from jax.experimental.pallas import tpu as pltpu
from jax.experimental.pallas import tpu_sc as plsc

_N_T = 16777216
_L = 16
_NW = 32
_PER_W = _N_T // _NW
_CHUNK = 16384
_NCHUNK = _PER_W // _CHUNK
_VPC = _CHUNK // _L

_INV_DX = 32.0
_DX = 1.0 / 32.0
_W12 = 0.1875
_W = 0.09375

_mesh = plsc.VectorSubcoreMesh(core_axis_name="c", subcore_axis_name="s")


@functools.partial(
    pl.kernel,
    out_type=jax.ShapeDtypeStruct((_N_T,), jnp.float32),
    mesh=_mesh,
    compiler_params=pltpu.CompilerParams(needs_layout_passes=False),
    scratch_types=[
        pltpu.VMEM((40,), jnp.float32),
        pltpu.VMEM((64,), jnp.float32),
        pltpu.VMEM((48,), jnp.float32),
        pltpu.VMEM((32,), jnp.float32),
        pltpu.VMEM((32,), jnp.float32),
        pltpu.VMEM((32,), jnp.float32),
        pltpu.VMEM((32,), jnp.float32),
        pltpu.VMEM((_CHUNK,), jnp.float32),
        pltpu.VMEM((_CHUNK,), jnp.float32),
        pltpu.VMEM((_CHUNK,), jnp.float32),
        pltpu.VMEM((_CHUNK,), jnp.float32),
        pltpu.SemaphoreType.DMA,
        pltpu.SemaphoreType.DMA,
        pltpu.SemaphoreType.DMA,
        pltpu.SemaphoreType.DMA,
    ],
)
def _pchip_sc(t_hbm, y_hbm, out_hbm, y_v, slp_v, d_v, c0, c1, c2, c3,
              tb0, tb1, ob0, ob1, sin0, sin1, sout0, sout1):
    wid = lax.axis_index("s") * 2 + lax.axis_index("c")
    base = wid * _PER_W
    pltpu.async_copy(t_hbm.at[pl.ds(base, _CHUNK)], tb0, sin0)
    pltpu.sync_copy(y_hbm, y_v.at[pl.ds(0, 33)])

    ya = y_v[pl.ds(0, _L)]
    yb = y_v[pl.ds(1, _L)]
    yc = y_v[pl.ds(16, _L)]
    yd = y_v[pl.ds(17, _L)]
    sa = (yb - ya) * _INV_DX
    sb = (yd - yc) * _INV_DX
    slp_v[pl.ds(0, _L)] = sa
    slp_v[pl.ds(1, _L)] = sa
    slp_v[pl.ds(18, _L)] = sb
    slp_v[pl.ds(17, _L)] = sb

    lane = lax.iota(jnp.int32, _L)
    for k in range(3):
        s0 = slp_v[pl.ds(16 * k, _L)]
        s1 = slp_v[pl.ds(16 * k + 1, _L)]
        cond = (s0 * s1) > 0.0
        safe0 = jnp.where(cond, s0, 1.0)
        safe1 = jnp.where(cond, s1, 1.0)
        hm = jnp.where(cond, _W12 / (_W / safe0 + _W / safe1), 0.0)
        if k == 0:
            hm = jnp.where(lane == 0, s1, hm)
        if k == 2:
            hm = jnp.where(lane == 0, s0, hm)
        d_v[pl.ds(16 * k, _L)] = hm

    for h in range(2):
        y0 = y_v[pl.ds(16 * h, _L)]
        y1 = y_v[pl.ds(16 * h + 1, _L)]
        d0 = d_v[pl.ds(16 * h, _L)]
        d1 = d_v[pl.ds(16 * h + 1, _L)]
        m0 = d0 * _DX
        m1 = d1 * _DX
        c0[pl.ds(16 * h, _L)] = y0
        c1[pl.ds(16 * h, _L)] = m0
        c2[pl.ds(16 * h, _L)] = 3.0 * (y1 - y0) - 2.0 * m0 - m1
        c3[pl.ds(16 * h, _L)] = 2.0 * (y0 - y1) + m0 + m1

    def compute(tbuf, obuf):
        @plsc.parallel_loop(0, _VPC, 1, unroll=12)
        def _(j):
            tv = tbuf[pl.ds(j * _L, _L)]
            s = tv * _INV_DX
            idx = s.astype(jnp.int32)
            u = s - idx.astype(jnp.float32)
            g0 = plsc.load_gather(c0, [idx])
            g1 = plsc.load_gather(c1, [idx])
            g2 = plsc.load_gather(c2, [idx])
            g3 = plsc.load_gather(c3, [idx])
            obuf[pl.ds(j * _L, _L)] = g0 + u * (g1 + u * (g2 + u * g3))

    def pipe_body(i, _):
        off_a = base + (2 * i) * _CHUNK
        off_b = off_a + _CHUNK
        pltpu.async_copy(t_hbm.at[pl.ds(off_b, _CHUNK)], tb1, sin1)
        pltpu.make_async_copy(t_hbm.at[pl.ds(off_a, _CHUNK)], tb0, sin0).wait()

        @pl.when(i > 0)
        def _():
            pltpu.make_async_copy(ob0, out_hbm.at[pl.ds(off_a, _CHUNK)],
                                  sout0).wait()
        compute(tb0, ob0)
        pltpu.async_copy(ob0, out_hbm.at[pl.ds(off_a, _CHUNK)], sout0)

        @pl.when(i + 1 < _NCHUNK // 2)
        def _():
            pltpu.async_copy(t_hbm.at[pl.ds(off_b + _CHUNK, _CHUNK)], tb0,
                             sin0)
        pltpu.make_async_copy(t_hbm.at[pl.ds(off_b, _CHUNK)], tb1, sin1).wait()

        @pl.when(i > 0)
        def _():
            pltpu.make_async_copy(ob1, out_hbm.at[pl.ds(off_b, _CHUNK)],
                                  sout1).wait()
        compute(tb1, ob1)
        pltpu.async_copy(ob1, out_hbm.at[pl.ds(off_b, _CHUNK)], sout1)
        return ()

    lax.fori_loop(0, _NCHUNK // 2, pipe_body, ())
    pltpu.make_async_copy(ob0, out_hbm.at[pl.ds(base, _CHUNK)], sout0).wait()
    pltpu.make_async_copy(ob1, out_hbm.at[pl.ds(base, _CHUNK)], sout1).wait()


def kernel(t, x, y):
    del x
    return _pchip_sc(t, y)


# speedup vs baseline: 1.0274x; 1.0274x over previous
import functools

import jax
import jax.numpy as jnp
from jax import lax
from jax.experimental import pallas as pl
from jax.experimental.pallas import tpu as pltpu
from jax.experimental.pallas import tpu_sc as plsc

_N_T = 16777216
_L = 16
_NW = 32
_PER_W = _N_T // _NW
_CHUNK = 16384
_NCHUNK = _PER_W // _CHUNK
_VPC = _CHUNK // _L

_INV_DX = 32.0
_DX = 1.0 / 32.0
_W12 = 0.1875
_W = 0.09375

_mesh = plsc.VectorSubcoreMesh(core_axis_name="c", subcore_axis_name="s")


@functools.partial(
    pl.kernel,
    out_type=jax.ShapeDtypeStruct((_N_T,), jnp.float32),
    mesh=_mesh,
    compiler_params=pltpu.CompilerParams(needs_layout_passes=False),
    scratch_types=[
        pltpu.VMEM((40,), jnp.float32),
        pltpu.VMEM((64,), jnp.float32),
        pltpu.VMEM((48,), jnp.float32),
        pltpu.VMEM((32,), jnp.float32),
        pltpu.VMEM((32,), jnp.float32),
        pltpu.VMEM((32,), jnp.float32),
        pltpu.VMEM((32,), jnp.float32),
        pltpu.VMEM((_CHUNK,), jnp.float32),
        pltpu.VMEM((_CHUNK,), jnp.float32),
        pltpu.VMEM((_CHUNK,), jnp.float32),
        pltpu.VMEM((_CHUNK,), jnp.float32),
        pltpu.SemaphoreType.DMA,
        pltpu.SemaphoreType.DMA,
        pltpu.SemaphoreType.DMA,
        pltpu.SemaphoreType.DMA,
    ],
)
def _pchip_sc(t_hbm, y_hbm, out_hbm, y_v, slp_v, d_v, c0, c1, c2, c3,
              tb0, tb1, ob0, ob1, sin0, sin1, sout0, sout1):
    wid = lax.axis_index("s") * 2 + lax.axis_index("c")
    base = wid * _PER_W
    pltpu.async_copy(t_hbm.at[pl.ds(base, _CHUNK)], tb0, sin0)
    pltpu.sync_copy(y_hbm, y_v.at[pl.ds(0, 33)])

    ya = y_v[pl.ds(0, _L)]
    yb = y_v[pl.ds(1, _L)]
    yc = y_v[pl.ds(16, _L)]
    yd = y_v[pl.ds(17, _L)]
    sa = (yb - ya) * _INV_DX
    sb = (yd - yc) * _INV_DX
    slp_v[pl.ds(0, _L)] = sa
    slp_v[pl.ds(1, _L)] = sa
    slp_v[pl.ds(18, _L)] = sb
    slp_v[pl.ds(17, _L)] = sb

    lane = lax.iota(jnp.int32, _L)
    for k in range(3):
        s0 = slp_v[pl.ds(16 * k, _L)]
        s1 = slp_v[pl.ds(16 * k + 1, _L)]
        cond = (s0 * s1) > 0.0
        safe0 = jnp.where(cond, s0, 1.0)
        safe1 = jnp.where(cond, s1, 1.0)
        hm = jnp.where(cond, _W12 / (_W / safe0 + _W / safe1), 0.0)
        if k == 0:
            hm = jnp.where(lane == 0, s1, hm)
        if k == 2:
            hm = jnp.where(lane == 0, s0, hm)
        d_v[pl.ds(16 * k, _L)] = hm

    for h in range(2):
        y0 = y_v[pl.ds(16 * h, _L)]
        y1 = y_v[pl.ds(16 * h + 1, _L)]
        d0 = d_v[pl.ds(16 * h, _L)]
        d1 = d_v[pl.ds(16 * h + 1, _L)]
        m0 = d0 * _DX
        m1 = d1 * _DX
        c0[pl.ds(16 * h, _L)] = y0
        c1[pl.ds(16 * h, _L)] = m0
        c2[pl.ds(16 * h, _L)] = 3.0 * (y1 - y0) - 2.0 * m0 - m1
        c3[pl.ds(16 * h, _L)] = 2.0 * (y0 - y1) + m0 + m1

    def compute(tbuf, obuf):
        @plsc.parallel_loop(0, _VPC, 1, unroll=16)
        def _(j):
            tv = tbuf[pl.ds(j * _L, _L)]
            s = tv * _INV_DX
            idx = s.astype(jnp.int32)
            u = s - idx.astype(jnp.float32)
            g0 = plsc.load_gather(c0, [idx])
            g1 = plsc.load_gather(c1, [idx])
            g2 = plsc.load_gather(c2, [idx])
            g3 = plsc.load_gather(c3, [idx])
            obuf[pl.ds(j * _L, _L)] = g0 + u * (g1 + u * (g2 + u * g3))

    def pipe_body(i, _):
        off_a = base + (2 * i) * _CHUNK
        off_b = off_a + _CHUNK
        pltpu.async_copy(t_hbm.at[pl.ds(off_b, _CHUNK)], tb1, sin1)
        pltpu.make_async_copy(t_hbm.at[pl.ds(off_a, _CHUNK)], tb0, sin0).wait()

        @pl.when(i > 0)
        def _():
            pltpu.make_async_copy(ob0, out_hbm.at[pl.ds(off_a, _CHUNK)],
                                  sout0).wait()
        compute(tb0, ob0)
        pltpu.async_copy(ob0, out_hbm.at[pl.ds(off_a, _CHUNK)], sout0)

        @pl.when(i + 1 < _NCHUNK // 2)
        def _():
            pltpu.async_copy(t_hbm.at[pl.ds(off_b + _CHUNK, _CHUNK)], tb0,
                             sin0)
        pltpu.make_async_copy(t_hbm.at[pl.ds(off_b, _CHUNK)], tb1, sin1).wait()

        @pl.when(i > 0)
        def _():
            pltpu.make_async_copy(ob1, out_hbm.at[pl.ds(off_b, _CHUNK)],
                                  sout1).wait()
        compute(tb1, ob1)
        pltpu.async_copy(ob1, out_hbm.at[pl.ds(off_b, _CHUNK)], sout1)
        return ()

    lax.fori_loop(0, _NCHUNK // 2, pipe_body, ())
    pltpu.make_async_copy(ob0, out_hbm.at[pl.ds(base, _CHUNK)], sout0).wait()
    pltpu.make_async_copy(ob1, out_hbm.at[pl.ds(base, _CHUNK)], sout1).wait()


def kernel(t, x, y):
    del x
    return _pchip_sc(t, y)
